# baseline (speedup 1.0000x reference)
import functools

import jax
import jax.numpy as jnp
from jax import lax
from jax.experimental import pallas as pl
from jax.experimental.pallas import tpu as pltpu
from jax.experimental.pallas import tpu_sc as plsc

D_MODEL = 64
SCALE = 8.0
LANES = 16
GB = 16


@functools.partial(jax.jit, static_argnames=("n_groups", "n_workers"))
def _embed_sc(x, table, *, n_groups, n_workers):
    batch, seq = x.shape
    bpw = batch // n_workers
    info = plsc.get_sparse_core_info()
    nc, ns = info.num_cores, info.num_subcores
    assert nc * ns == n_workers
    mesh = plsc.VectorSubcoreMesh(core_axis_name="c", subcore_axis_name="s")

    @functools.partial(
        pl.kernel,
        mesh=mesh,
        compiler_params=pltpu.CompilerParams(use_tc_tiling_on_sc=False),
        out_type=jax.ShapeDtypeStruct((batch, seq, D_MODEL), jnp.float32),
        scratch_types=[
            pltpu.VMEM((bpw, seq), jnp.int32),
            pltpu.VMEM((GB, seq, D_MODEL), jnp.float32),
            pltpu.VMEM((GB, seq, D_MODEL), jnp.float32),
            pltpu.SemaphoreType.DMA,
            pltpu.SemaphoreType.DMA,
        ],
    )
    def body(table_hbm, x_hbm, out_hbm, idx_v, rows_a, rows_b, sem_a, sem_b):
        wid = lax.axis_index("s") * nc + lax.axis_index("c")
        row0 = wid * bpw
        pltpu.sync_copy(x_hbm.at[pl.ds(row0, bpw)], idx_v)
        bufs = (rows_a, rows_b)
        sems = (sem_a, sem_b)

        def fire(g, b):
            descs = []
            for i in range(GB):
                idx_sl = idx_v.at[g * GB + i]
                descs.append(
                    pltpu.async_copy(table_hbm.at[idx_sl], bufs[b].at[i], sems[b])
                )
            return descs

        def scale(b):
            rows = bufs[b]
            for i in range(GB):
                def scale_rows(r, _):
                    for j in range(D_MODEL // LANES):
                        sl = pl.ds(j * LANES, LANES)
                        rows[i, r, sl] = rows[i, r, sl] * SCALE
                    return 0

                lax.fori_loop(0, seq, scale_rows, 0)

        in_flight = {0: fire(0, 0)}
        for g in range(n_groups):
            b = g & 1
            if g + 1 < n_groups:
                in_flight[g + 1] = fire(g + 1, 1 - b)
            for d in in_flight.pop(g):
                d.wait()
            scale(b)
            pltpu.sync_copy(bufs[b], out_hbm.at[pl.ds(row0 + g * GB, GB)])

    return body(table, x)


def kernel(x, table):
    n_workers = 32
    batch = x.shape[0]
    assert batch % (n_workers * GB) == 0
    n_groups = batch // (n_workers * GB)
    return _embed_sc(x, table, n_groups=n_groups, n_workers=n_workers)
